# baseline (speedup 1.0000x reference)
import jax
import jax.numpy as jnp
from jax.experimental import pallas as pl
from jax.experimental.pallas import tpu as pltpu

SEQ_BLK = 64
NCHAIN = 2
UNROLL = 8


def _rnn_kernel(xs_ref, wcat_ref, wout_ref, bout_ref, out_ref, buf_ref):
    j = pl.program_id(0)
    nseq = pl.num_programs(0)
    cb = buf_ref.shape[1]

    @pl.when(j == 0)
    def _():
        buf_ref[...] = jnp.zeros_like(buf_ref)
        for c in range(NCHAIN):
            buf_ref[c, :, 127:128] = jnp.ones((cb, 1), jnp.float32)

    for c in range(NCHAIN):
        buf_ref[c, :, 0:64] = jnp.swapaxes(
            xs_ref[0, :, c * cb:(c + 1) * cb], 0, 1)

    wcat = wcat_ref[...]

    def step(i, _):
        for u in range(UNROLL):
            t = i * UNROLL + u
            zs = [jnp.dot(buf_ref[c], wcat, preferred_element_type=jnp.float32)
                  for c in range(NCHAIN)]
            tn = jnp.minimum(t + 1, SEQ_BLK - 1)
            for c in range(NCHAIN):
                buf_ref[c, :, 0:64] = jnp.swapaxes(
                    xs_ref[tn, :, c * cb:(c + 1) * cb], 0, 1)
            for c in range(NCHAIN):
                buf_ref[c, :, 128:256] = jnp.tanh(zs[c][:, 0:128])
        return 0

    jax.lax.fori_loop(0, SEQ_BLK // UNROLL, step, 0)

    @pl.when(j == nseq - 1)
    def _():
        wout = wout_ref[...]
        bout = bout_ref[...]
        for c in range(NCHAIN):
            h = buf_ref[c, :, 128:256]
            out_ref[c * cb:(c + 1) * cb, :] = (
                jnp.sum(h * wout, axis=1, keepdims=True) + bout)


def kernel(xs, W_ih, W_hh, b_ih, b_hh, W_out, b_out):
    seq, batch, inpt = xs.shape
    hidden = W_hh.shape[0]
    cb = batch // NCHAIN
    wcat = jnp.zeros((2 * hidden, 2 * hidden), dtype=jnp.float32)
    wcat = wcat.at[0:inpt, 0:hidden].set(W_ih.T)
    wcat = wcat.at[hidden - 1, 0:hidden].set(b_ih + b_hh)
    wcat = wcat.at[hidden:, 0:hidden].set(W_hh.T)
    wout = W_out.reshape(1, hidden)
    bout = b_out.reshape(1, 1)
    xs_t = jnp.swapaxes(xs, 1, 2)
    return pl.pallas_call(
        _rnn_kernel,
        grid=(seq // SEQ_BLK,),
        in_specs=[
            pl.BlockSpec((SEQ_BLK, inpt, batch), lambda j: (j, 0, 0)),
            pl.BlockSpec((2 * hidden, 2 * hidden), lambda j: (0, 0)),
            pl.BlockSpec((1, hidden), lambda j: (0, 0)),
            pl.BlockSpec((1, 1), lambda j: (0, 0)),
        ],
        out_specs=pl.BlockSpec((batch, 1), lambda j: (0, 0)),
        out_shape=jax.ShapeDtypeStruct((batch, 1), xs.dtype),
        scratch_shapes=[pltpu.VMEM((NCHAIN, cb, 2 * hidden), jnp.float32)],
        compiler_params=pltpu.CompilerParams(
            dimension_semantics=("arbitrary",),
        ),
    )(xs_t, wcat, wout, bout)


# speedup vs baseline: 1.0283x; 1.0283x over previous
import jax
import jax.numpy as jnp
from jax.experimental import pallas as pl
from jax.experimental.pallas import tpu as pltpu

SEQ_BLK = 64
UNROLL = 8


def _rnn_kernel(xs_ref, wcat_ref, wout_ref, bout_ref, out_ref,
                xstage_ref, h_ref):
    j = pl.program_id(0)
    nseq = pl.num_programs(0)
    batch = xstage_ref.shape[0]

    @pl.when(j == 0)
    def _():
        xstage_ref[...] = jnp.zeros_like(xstage_ref)
        xstage_ref[:, 127:128] = jnp.ones((batch, 1), jnp.float32)
        h_ref[...] = jnp.zeros_like(h_ref)

    xstage_ref[:, 0:64] = jnp.swapaxes(xs_ref[0], 0, 1)

    wcat = wcat_ref[...]

    def step(i, h):
        for u in range(UNROLL):
            t = i * UNROLL + u
            lhs = jnp.concatenate([h, xstage_ref[...]], axis=1)
            z = jnp.dot(lhs, wcat, preferred_element_type=jnp.float32)
            tn = jnp.minimum(t + 1, SEQ_BLK - 1)
            xstage_ref[:, 0:64] = jnp.swapaxes(xs_ref[tn], 0, 1)
            h = jnp.tanh(z[:, 0:128])
        return h

    h = jax.lax.fori_loop(0, SEQ_BLK // UNROLL, step, h_ref[...])
    h_ref[...] = h

    @pl.when(j == nseq - 1)
    def _():
        out_ref[...] = (jnp.sum(h * wout_ref[...], axis=1, keepdims=True)
                        + bout_ref[...])


def kernel(xs, W_ih, W_hh, b_ih, b_hh, W_out, b_out):
    seq, batch, inpt = xs.shape
    hidden = W_hh.shape[0]
    wcat = jnp.zeros((2 * hidden, 2 * hidden), dtype=jnp.float32)
    wcat = wcat.at[0:hidden, 0:hidden].set(W_hh.T)
    wcat = wcat.at[hidden:hidden + inpt, 0:hidden].set(W_ih.T)
    wcat = wcat.at[2 * hidden - 1, 0:hidden].set(b_ih + b_hh)
    wout = W_out.reshape(1, hidden)
    bout = b_out.reshape(1, 1)
    xs_t = jnp.swapaxes(xs, 1, 2)
    return pl.pallas_call(
        _rnn_kernel,
        grid=(seq // SEQ_BLK,),
        in_specs=[
            pl.BlockSpec((SEQ_BLK, inpt, batch), lambda j: (j, 0, 0)),
            pl.BlockSpec((2 * hidden, 2 * hidden), lambda j: (0, 0)),
            pl.BlockSpec((1, hidden), lambda j: (0, 0)),
            pl.BlockSpec((1, 1), lambda j: (0, 0)),
        ],
        out_specs=pl.BlockSpec((batch, 1), lambda j: (0, 0)),
        out_shape=jax.ShapeDtypeStruct((batch, 1), xs.dtype),
        scratch_shapes=[
            pltpu.VMEM((batch, hidden), jnp.float32),
            pltpu.VMEM((batch, hidden), jnp.float32),
        ],
        compiler_params=pltpu.CompilerParams(
            dimension_semantics=("arbitrary",),
        ),
    )(xs_t, wcat, wout, bout)


# speedup vs baseline: 1.1126x; 1.0820x over previous
import jax
import jax.numpy as jnp
from jax.experimental import pallas as pl
from jax.experimental.pallas import tpu as pltpu

SEQ_BLK = 64
UNROLL = 16


def _rnn_kernel(xs_ref, wcat_ref, wout_ref, bout_ref, out_ref,
                xstage_ref, h_ref):
    j = pl.program_id(0)
    nseq = pl.num_programs(0)
    batch = xstage_ref.shape[0]

    @pl.when(j == 0)
    def _():
        xstage_ref[...] = jnp.zeros_like(xstage_ref)
        xstage_ref[:, 127:128] = jnp.ones((batch, 1), jnp.float32)
        h_ref[...] = jnp.zeros_like(h_ref)

    xstage_ref[:, 0:64] = jnp.swapaxes(xs_ref[0], 0, 1)

    wcat = wcat_ref[...]

    def step(i, h):
        for u in range(UNROLL):
            t = i * UNROLL + u
            lhs = jnp.concatenate([h, xstage_ref[...]], axis=1)
            z = jnp.dot(lhs, wcat, preferred_element_type=jnp.float32)
            tn = jnp.minimum(t + 1, SEQ_BLK - 1)
            xstage_ref[:, 0:64] = jnp.swapaxes(xs_ref[tn], 0, 1)
            h = jnp.tanh(z[:, 0:128])
        return h

    h = jax.lax.fori_loop(0, SEQ_BLK // UNROLL, step, h_ref[...])
    h_ref[...] = h

    @pl.when(j == nseq - 1)
    def _():
        out_ref[...] = (jnp.sum(h * wout_ref[...], axis=1, keepdims=True)
                        + bout_ref[...])


def kernel(xs, W_ih, W_hh, b_ih, b_hh, W_out, b_out):
    seq, batch, inpt = xs.shape
    hidden = W_hh.shape[0]
    wcat = jnp.zeros((2 * hidden, 2 * hidden), dtype=jnp.float32)
    wcat = wcat.at[0:hidden, 0:hidden].set(W_hh.T)
    wcat = wcat.at[hidden:hidden + inpt, 0:hidden].set(W_ih.T)
    wcat = wcat.at[2 * hidden - 1, 0:hidden].set(b_ih + b_hh)
    wout = W_out.reshape(1, hidden)
    bout = b_out.reshape(1, 1)
    xs_t = jnp.swapaxes(xs, 1, 2)
    return pl.pallas_call(
        _rnn_kernel,
        grid=(seq // SEQ_BLK,),
        in_specs=[
            pl.BlockSpec((SEQ_BLK, inpt, batch), lambda j: (j, 0, 0)),
            pl.BlockSpec((2 * hidden, 2 * hidden), lambda j: (0, 0)),
            pl.BlockSpec((1, hidden), lambda j: (0, 0)),
            pl.BlockSpec((1, 1), lambda j: (0, 0)),
        ],
        out_specs=pl.BlockSpec((batch, 1), lambda j: (0, 0)),
        out_shape=jax.ShapeDtypeStruct((batch, 1), xs.dtype),
        scratch_shapes=[
            pltpu.VMEM((batch, hidden), jnp.float32),
            pltpu.VMEM((batch, hidden), jnp.float32),
        ],
        compiler_params=pltpu.CompilerParams(
            dimension_semantics=("arbitrary",),
        ),
    )(xs_t, wcat, wout, bout)


# speedup vs baseline: 1.1572x; 1.0401x over previous
import jax
import jax.numpy as jnp
from jax.experimental import pallas as pl
from jax.experimental.pallas import tpu as pltpu

SEQ_BLK = 64
UNROLL = 32


def _rnn_kernel(xs_ref, wcat_ref, wout_ref, bout_ref, out_ref,
                xstage_ref, h_ref):
    j = pl.program_id(0)
    nseq = pl.num_programs(0)
    batch = xstage_ref.shape[0]

    @pl.when(j == 0)
    def _():
        xstage_ref[...] = jnp.zeros_like(xstage_ref)
        xstage_ref[:, 127:128] = jnp.ones((batch, 1), jnp.float32)
        h_ref[...] = jnp.zeros_like(h_ref)

    xstage_ref[:, 0:64] = jnp.swapaxes(xs_ref[0], 0, 1)

    wcat = wcat_ref[...]

    def step(i, h):
        for u in range(UNROLL):
            t = i * UNROLL + u
            lhs = jnp.concatenate([h, xstage_ref[...]], axis=1)
            z = jnp.dot(lhs, wcat, preferred_element_type=jnp.float32)
            tn = jnp.minimum(t + 1, SEQ_BLK - 1)
            xstage_ref[:, 0:64] = jnp.swapaxes(xs_ref[tn], 0, 1)
            h = jnp.tanh(z[:, 0:128])
        return h

    h = jax.lax.fori_loop(0, SEQ_BLK // UNROLL, step, h_ref[...])
    h_ref[...] = h

    @pl.when(j == nseq - 1)
    def _():
        out_ref[...] = (jnp.sum(h * wout_ref[...], axis=1, keepdims=True)
                        + bout_ref[...])


def kernel(xs, W_ih, W_hh, b_ih, b_hh, W_out, b_out):
    seq, batch, inpt = xs.shape
    hidden = W_hh.shape[0]
    wcat = jnp.zeros((2 * hidden, 2 * hidden), dtype=jnp.float32)
    wcat = wcat.at[0:hidden, 0:hidden].set(W_hh.T)
    wcat = wcat.at[hidden:hidden + inpt, 0:hidden].set(W_ih.T)
    wcat = wcat.at[2 * hidden - 1, 0:hidden].set(b_ih + b_hh)
    wout = W_out.reshape(1, hidden)
    bout = b_out.reshape(1, 1)
    xs_t = jnp.swapaxes(xs, 1, 2)
    return pl.pallas_call(
        _rnn_kernel,
        grid=(seq // SEQ_BLK,),
        in_specs=[
            pl.BlockSpec((SEQ_BLK, inpt, batch), lambda j: (j, 0, 0)),
            pl.BlockSpec((2 * hidden, 2 * hidden), lambda j: (0, 0)),
            pl.BlockSpec((1, hidden), lambda j: (0, 0)),
            pl.BlockSpec((1, 1), lambda j: (0, 0)),
        ],
        out_specs=pl.BlockSpec((batch, 1), lambda j: (0, 0)),
        out_shape=jax.ShapeDtypeStruct((batch, 1), xs.dtype),
        scratch_shapes=[
            pltpu.VMEM((batch, hidden), jnp.float32),
            pltpu.VMEM((batch, hidden), jnp.float32),
        ],
        compiler_params=pltpu.CompilerParams(
            dimension_semantics=("arbitrary",),
        ),
    )(xs_t, wcat, wout, bout)


# speedup vs baseline: 1.2034x; 1.0399x over previous
import jax
import jax.numpy as jnp
from jax.experimental import pallas as pl
from jax.experimental.pallas import tpu as pltpu

SEQ_BLK = 64
UNROLL = 64


def _rnn_kernel(xs_ref, wcat_ref, wout_ref, bout_ref, out_ref,
                xstage_ref, h_ref):
    j = pl.program_id(0)
    nseq = pl.num_programs(0)
    batch = xstage_ref.shape[0]

    @pl.when(j == 0)
    def _():
        xstage_ref[...] = jnp.zeros_like(xstage_ref)
        xstage_ref[:, 127:128] = jnp.ones((batch, 1), jnp.float32)
        h_ref[...] = jnp.zeros_like(h_ref)

    xstage_ref[:, 0:64] = jnp.swapaxes(xs_ref[0], 0, 1)

    wcat = wcat_ref[...]

    def step(i, h):
        for u in range(UNROLL):
            t = i * UNROLL + u
            lhs = jnp.concatenate([h, xstage_ref[...]], axis=1)
            z = jnp.dot(lhs, wcat, preferred_element_type=jnp.float32)
            tn = jnp.minimum(t + 1, SEQ_BLK - 1)
            xstage_ref[:, 0:64] = jnp.swapaxes(xs_ref[tn], 0, 1)
            h = jnp.tanh(z[:, 0:128])
        return h

    h = jax.lax.fori_loop(0, SEQ_BLK // UNROLL, step, h_ref[...])
    h_ref[...] = h

    @pl.when(j == nseq - 1)
    def _():
        out_ref[...] = (jnp.sum(h * wout_ref[...], axis=1, keepdims=True)
                        + bout_ref[...])


def kernel(xs, W_ih, W_hh, b_ih, b_hh, W_out, b_out):
    seq, batch, inpt = xs.shape
    hidden = W_hh.shape[0]
    wcat = jnp.zeros((2 * hidden, 2 * hidden), dtype=jnp.float32)
    wcat = wcat.at[0:hidden, 0:hidden].set(W_hh.T)
    wcat = wcat.at[hidden:hidden + inpt, 0:hidden].set(W_ih.T)
    wcat = wcat.at[2 * hidden - 1, 0:hidden].set(b_ih + b_hh)
    wout = W_out.reshape(1, hidden)
    bout = b_out.reshape(1, 1)
    xs_t = jnp.swapaxes(xs, 1, 2)
    return pl.pallas_call(
        _rnn_kernel,
        grid=(seq // SEQ_BLK,),
        in_specs=[
            pl.BlockSpec((SEQ_BLK, inpt, batch), lambda j: (j, 0, 0)),
            pl.BlockSpec((2 * hidden, 2 * hidden), lambda j: (0, 0)),
            pl.BlockSpec((1, hidden), lambda j: (0, 0)),
            pl.BlockSpec((1, 1), lambda j: (0, 0)),
        ],
        out_specs=pl.BlockSpec((batch, 1), lambda j: (0, 0)),
        out_shape=jax.ShapeDtypeStruct((batch, 1), xs.dtype),
        scratch_shapes=[
            pltpu.VMEM((batch, hidden), jnp.float32),
            pltpu.VMEM((batch, hidden), jnp.float32),
        ],
        compiler_params=pltpu.CompilerParams(
            dimension_semantics=("arbitrary",),
        ),
    )(xs_t, wcat, wout, bout)
